# Initial kernel scaffold; baseline (speedup 1.0000x reference)
import jax
import jax.numpy as jnp
from jax.experimental import pallas as pl


def kernel(input_tokens, token_table, pos_table):
    raise NotImplementedError("write your pallas kernel here")


# speedup vs baseline: 1.5201x; 1.5201x over previous
import functools

import jax
import jax.numpy as jnp
from jax import lax
from jax.experimental import pallas as pl
from jax.experimental.pallas import tpu as pltpu
from jax.experimental.pallas import tpu_sc as plsc

BATCH = 4096
SEQ = 77
DIM = 768
LANES = 16
NCORES = 2
NSUB = 16
NW = NCORES * NSUB

CHUNK = 64
NB = BATCH // CHUNK
NTILES = SEQ * NB
TPW = NTILES // NW


def _sc_embed(idx_t, token_table, pos_table):
    mesh = plsc.VectorSubcoreMesh(core_axis_name="core", subcore_axis_name="sub")

    @functools.partial(
        pl.kernel,
        mesh=mesh,
        out_type=jax.ShapeDtypeStruct((BATCH, SEQ, DIM), jnp.float32),
        scratch_types=[
            pltpu.VMEM((CHUNK,), jnp.int32),
            pltpu.VMEM((CHUNK, DIM), jnp.float32),
            pltpu.VMEM((DIM,), jnp.float32),
            pltpu.SemaphoreType.DMA,
        ],
    )
    def k(idx_hbm, table_hbm, pos_hbm, out_hbm, idx_v, rows_v, pos_v, gsem):
        wid = lax.axis_index("sub") * NCORES + lax.axis_index("core")
        t0 = wid * TPW

        def tile_body(i, carry):
            t = t0 + i
            s_i = t // NB
            b0 = (t % NB) * CHUNK
            pltpu.sync_copy(pos_hbm.at[s_i], pos_v)
            pltpu.sync_copy(idx_hbm.at[s_i, pl.ds(b0, CHUNK)], idx_v)
            pltpu.async_copy(table_hbm.at[idx_v], rows_v, gsem).wait()
            pvecs = [pos_v[pl.ds(dd * LANES, LANES)] for dd in range(DIM // LANES)]

            def row_add(r, c):
                for dd in range(DIM // LANES):
                    sl = pl.ds(dd * LANES, LANES)
                    rows_v[r, sl] = rows_v[r, sl] + pvecs[dd]
                return c

            lax.fori_loop(0, CHUNK, row_add, 0)
            pltpu.sync_copy(rows_v, out_hbm.at[pl.ds(b0, CHUNK), s_i])
            return carry

        lax.fori_loop(0, TPW, tile_body, 0)

    return k(idx_t, token_table, pos_table)


def kernel(input_tokens, token_table, pos_table):
    idx_t = input_tokens.astype(jnp.int32).T
    return _sc_embed(idx_t, token_table, pos_table)


# speedup vs baseline: 2.0846x; 1.3714x over previous
import functools

import jax
import jax.numpy as jnp
from jax import lax
from jax.experimental import pallas as pl
from jax.experimental.pallas import tpu as pltpu
from jax.experimental.pallas import tpu_sc as plsc

BATCH = 4096
SEQ = 77
DIM = 768
LANES = 16
NCORES = 2
NSUB = 16
NW = NCORES * NSUB

CHUNK = 64
NB = BATCH // CHUNK
NTILES = SEQ * NB
TPW = NTILES // NW
NBUF = 2


def _sc_embed(idx_t, token_table, pos_table):
    mesh = plsc.VectorSubcoreMesh(core_axis_name="core", subcore_axis_name="sub")

    scratch = []
    for _ in range(NBUF):
        scratch += [
            pltpu.VMEM((CHUNK,), jnp.int32),
            pltpu.VMEM((CHUNK, DIM), jnp.float32),
            pltpu.VMEM((DIM,), jnp.float32),
            pltpu.SemaphoreType.DMA,
            pltpu.SemaphoreType.DMA,
        ]

    @functools.partial(
        pl.kernel,
        mesh=mesh,
        out_type=jax.ShapeDtypeStruct((BATCH, SEQ, DIM), jnp.float32),
        scratch_types=scratch,
    )
    def k(idx_hbm, table_hbm, pos_hbm, out_hbm, *bufs):
        wid = lax.axis_index("sub") * NCORES + lax.axis_index("core")
        t0 = wid * TPW
        rings = [tuple(bufs[5 * b:5 * b + 5]) for b in range(NBUF)]

        def coords(kk):
            t = t0 + kk
            s_i = t // NB
            b0 = (t % NB) * CHUNK
            return s_i, b0

        def start_gather(kk, idx_v, rows_v, pos_v, gsem):
            s_i, b0 = coords(kk)
            pltpu.sync_copy(pos_hbm.at[s_i], pos_v)
            pltpu.sync_copy(idx_hbm.at[s_i, pl.ds(b0, CHUNK)], idx_v)
            pltpu.async_copy(table_hbm.at[idx_v], rows_v, gsem)

        def wait_gather(idx_v, rows_v, gsem):
            pltpu.make_async_copy(table_hbm.at[idx_v], rows_v, gsem).wait()

        def wait_store(kk, rows_v, ssem):
            s_i, b0 = coords(kk)
            pltpu.make_async_copy(
                rows_v, out_hbm.at[pl.ds(b0, CHUNK), s_i], ssem).wait()

        for b in range(NBUF):
            idx_v, rows_v, pos_v, gsem, _ = rings[b]
            start_gather(b, idx_v, rows_v, pos_v, gsem)

        def outer(jj, carry):
            for b in range(NBUF):
                idx_v, rows_v, pos_v, gsem, ssem = rings[b]
                kk = jj * NBUF + b
                s_i, b0 = coords(kk)
                wait_gather(idx_v, rows_v, gsem)
                pvecs = [pos_v[pl.ds(dd * LANES, LANES)]
                         for dd in range(DIM // LANES)]

                def row_add(r, c):
                    for dd in range(DIM // LANES):
                        sl = pl.ds(dd * LANES, LANES)
                        rows_v[r, sl] = rows_v[r, sl] + pvecs[dd]
                    return c

                lax.fori_loop(0, CHUNK, row_add, 0)
                pltpu.async_copy(
                    rows_v, out_hbm.at[pl.ds(b0, CHUNK), s_i], ssem)

                @pl.when(kk + NBUF < TPW)
                def _prep_next():
                    kn = kk + NBUF
                    s_n, b0_n = coords(kn)
                    pltpu.sync_copy(pos_hbm.at[s_n], pos_v)
                    pltpu.sync_copy(idx_hbm.at[s_n, pl.ds(b0_n, CHUNK)], idx_v)
                    wait_store(kk, rows_v, ssem)
                    pltpu.async_copy(table_hbm.at[idx_v], rows_v, gsem)
            return carry

        lax.fori_loop(0, TPW // NBUF, outer, 0)

        for b in range(NBUF):
            _, rows_v, _, _, ssem = rings[b]
            wait_store(TPW - NBUF + b, rows_v, ssem)

    return k(idx_t, token_table, pos_table)


def kernel(input_tokens, token_table, pos_table):
    idx_t = input_tokens.astype(jnp.int32).T
    return _sc_embed(idx_t, token_table, pos_table)


# speedup vs baseline: 4.0167x; 1.9269x over previous
import functools

import jax
import jax.numpy as jnp
from jax import lax
from jax.experimental import pallas as pl
from jax.experimental.pallas import tpu as pltpu
from jax.experimental.pallas import tpu_sc as plsc

BATCH = 4096
SEQ = 77
DIM = 768
LANES = 16
NCORES = 2
NSUB = 16
NW = NCORES * NSUB

CHUNK = 64
NB = BATCH // CHUNK
NTILES = SEQ * NB
TPW = NTILES // NW
NBUF = 2


def _sc_embed(idx_t, token_table, pos_table):
    mesh = plsc.VectorSubcoreMesh(core_axis_name="core", subcore_axis_name="sub")

    scratch = []
    for _ in range(NBUF):
        scratch += [
            pltpu.VMEM((CHUNK,), jnp.int32),
            pltpu.VMEM((CHUNK, DIM), jnp.float32),
            pltpu.VMEM((DIM,), jnp.float32),
            pltpu.SemaphoreType.DMA,
            pltpu.SemaphoreType.DMA,
        ]

    @functools.partial(
        pl.kernel,
        mesh=mesh,
        out_type=jax.ShapeDtypeStruct((SEQ, BATCH, DIM), jnp.float32),
        scratch_types=scratch,
    )
    def k(idx_hbm, table_hbm, pos_hbm, out_hbm, *bufs):
        wid = lax.axis_index("sub") * NCORES + lax.axis_index("core")
        t0 = wid * TPW
        rings = [tuple(bufs[5 * b:5 * b + 5]) for b in range(NBUF)]

        def coords(kk):
            t = t0 + kk
            s_i = t // NB
            b0 = (t % NB) * CHUNK
            return s_i, b0

        def start_gather(kk, idx_v, rows_v, pos_v, gsem):
            s_i, b0 = coords(kk)
            pltpu.sync_copy(pos_hbm.at[s_i], pos_v)
            pltpu.sync_copy(idx_hbm.at[s_i, pl.ds(b0, CHUNK)], idx_v)
            pltpu.async_copy(table_hbm.at[idx_v], rows_v, gsem)

        def wait_gather(idx_v, rows_v, gsem):
            pltpu.make_async_copy(table_hbm.at[idx_v], rows_v, gsem).wait()

        def wait_store(kk, rows_v, ssem):
            s_i, b0 = coords(kk)
            pltpu.make_async_copy(
                rows_v, out_hbm.at[s_i, pl.ds(b0, CHUNK)], ssem).wait()

        for b in range(NBUF):
            idx_v, rows_v, pos_v, gsem, _ = rings[b]
            start_gather(b, idx_v, rows_v, pos_v, gsem)

        def outer(jj, carry):
            for b in range(NBUF):
                idx_v, rows_v, pos_v, gsem, ssem = rings[b]
                kk = jj * NBUF + b
                s_i, b0 = coords(kk)
                wait_gather(idx_v, rows_v, gsem)
                pvecs = [pos_v[pl.ds(dd * LANES, LANES)]
                         for dd in range(DIM // LANES)]

                def row_add(r, c):
                    for dd in range(DIM // LANES):
                        sl = pl.ds(dd * LANES, LANES)
                        rows_v[r, sl] = rows_v[r, sl] + pvecs[dd]
                    return c

                lax.fori_loop(0, CHUNK, row_add, 0)
                pltpu.async_copy(
                    rows_v, out_hbm.at[s_i, pl.ds(b0, CHUNK)], ssem)

                @pl.when(kk + NBUF < TPW)
                def _prep_next():
                    kn = kk + NBUF
                    s_n, b0_n = coords(kn)
                    pltpu.sync_copy(pos_hbm.at[s_n], pos_v)
                    pltpu.sync_copy(idx_hbm.at[s_n, pl.ds(b0_n, CHUNK)], idx_v)
                    wait_store(kk, rows_v, ssem)
                    pltpu.async_copy(table_hbm.at[idx_v], rows_v, gsem)
            return carry

        lax.fori_loop(0, TPW // NBUF, outer, 0)

        for b in range(NBUF):
            _, rows_v, _, _, ssem = rings[b]
            wait_store(TPW - NBUF + b, rows_v, ssem)

    return k(idx_t, token_table, pos_table)


def kernel(input_tokens, token_table, pos_table):
    idx_t = input_tokens.astype(jnp.int32).T
    out = _sc_embed(idx_t, token_table, pos_table)
    return jnp.transpose(out, (1, 0, 2))
